# baseline (speedup 1.0000x reference)
import functools

import jax
import jax.numpy as jnp
from jax import lax
from jax.experimental import pallas as pl
from jax.experimental.pallas import tpu as pltpu
from jax.experimental.pallas import tpu_sc as plsc

_LAMBDA_2 = 0.01

_B = 16384
_D = 64
_L = 16
_NC = 2
_NS = 16
_NW = _NC * _NS
_BPW = _B // _NW
_CHUNK = 128
_NCH = _BPW // _CHUNK

_mesh = plsc.VectorSubcoreMesh(core_axis_name="c", subcore_axis_name="s")


@functools.partial(
    pl.kernel,
    mesh=_mesh,
    compiler_params=pltpu.CompilerParams(use_tc_tiling_on_sc=False),
    out_type=[
        jax.ShapeDtypeStruct((_NW, _L), jnp.float32),
        jax.ShapeDtypeStruct((_NW, _L), jnp.float32),
    ],
    scratch_types=[
        pltpu.VMEM((_NCH, _CHUNK), jnp.int32),
        pltpu.VMEM((_NCH, _CHUNK), jnp.int32),
        pltpu.VMEM((_BPW,), jnp.float32),
        pltpu.VMEM((_NCH, _CHUNK, _D), jnp.float32),
        pltpu.VMEM((_NCH, _CHUNK, _D), jnp.float32),
        pltpu.VMEM((_L,), jnp.float32),
        pltpu.VMEM((_L,), jnp.float32),
        pltpu.SemaphoreType.DMA,
    ],
)
def _phi_partials(w_hbm, coo_hbm, idx1_hbm, idx2_hbm, out_sq_hbm, out_abs_hbm,
                  idx1_v, idx2_v, coo_v, rows1_v, rows2_v, sq_v, abs_v, sem):
    wid = lax.axis_index("s") * _NC + lax.axis_index("c")
    base = wid * _BPW

    pltpu.sync_copy(idx1_hbm.at[pl.ds(wid * _NCH, _NCH)], idx1_v)
    pltpu.sync_copy(idx2_hbm.at[pl.ds(wid * _NCH, _NCH)], idx2_v)
    pltpu.sync_copy(coo_hbm.at[pl.ds(base, _BPW)], coo_v)

    copies = []
    for j in range(_NCH):
        copies.append(pltpu.async_copy(w_hbm.at[idx1_v.at[j]], rows1_v.at[j], sem))
        copies.append(pltpu.async_copy(w_hbm.at[idx2_v.at[j]], rows2_v.at[j], sem))
    for cp in copies:
        cp.wait()

    zero = jnp.zeros((_L,), jnp.float32)
    acc_sq, acc_abs = zero, zero
    for j in range(_NCH):
        def body(g, carry, j=j):
            a_sq, a_abs = carry
            cvec = coo_v[pl.ds(j * _CHUNK + g * _L, _L)]
            for l in range(_L):
                cb = jnp.full((_L,), cvec[l], dtype=jnp.float32)
                r = g * _L + l
                for k in range(_D // _L):
                    a = rows1_v[j, r, pl.ds(k * _L, _L)]
                    b = rows2_v[j, r, pl.ds(k * _L, _L)]
                    d = cb - a * b
                    a_sq = a_sq + d * d
                    a_abs = a_abs + jnp.abs(a) + jnp.abs(b)
            return a_sq, a_abs
        acc_sq, acc_abs = lax.fori_loop(0, _CHUNK // _L, body, (acc_sq, acc_abs))

    sq_v[...] = acc_sq
    abs_v[...] = acc_abs
    pltpu.sync_copy(sq_v, out_sq_hbm.at[wid])
    pltpu.sync_copy(abs_v, out_abs_hbm.at[wid])


def kernel(w, cooccur, feature_idx1, feature_idx2):
    idx1 = feature_idx1.astype(jnp.int32).reshape(_NW * _NCH, _CHUNK)
    idx2 = feature_idx2.astype(jnp.int32).reshape(_NW * _NCH, _CHUNK)
    coo = cooccur.reshape(_B)
    sq, ab = _phi_partials(w, coo, idx1, idx2)
    return jnp.sqrt(jnp.sum(sq)) + (_LAMBDA_2 / 2.0) * jnp.sum(ab)
